# baseline (speedup 1.0000x reference)
import jax
import jax.numpy as jnp
import numpy as np
from jax import lax
from jax.experimental import pallas as pl
from jax.experimental.pallas import tpu as pltpu
from jax.experimental.pallas import tpu_sc as plsc

N = 10000
D = 128
E = 320000

NSC = 2
NT = 16
NW = NSC * NT
EPT = E // NW
K = 80
CH = EPT // K
N2 = 10240
K2 = 128
CH2 = 79
E2 = NW * CH2 * K2
RPT = N2 // NT
ZR = 128
LANES = 16
R = 2000


def _tc_transform():
    def body(x_ref, wl_ref, wr_ref, b_ref, xp_ref, r_ref):
        xb = x_ref[...]
        dn = (((1,), (1,)), ((), ()))
        xp_ref[...] = lax.dot_general(xb, wl_ref[...], dn,
                                      preferred_element_type=jnp.float32)
        r_ref[...] = lax.dot_general(xb, wr_ref[...], dn,
                                     preferred_element_type=jnp.float32) + b_ref[...]

    return pl.pallas_call(
        body,
        grid=(N // R,),
        in_specs=[
            pl.BlockSpec((R, D), lambda i: (i, 0)),
            pl.BlockSpec((D, D), lambda i: (0, 0)),
            pl.BlockSpec((D, D), lambda i: (0, 0)),
            pl.BlockSpec((1, D), lambda i: (0, 0)),
        ],
        out_specs=[
            pl.BlockSpec((R, D), lambda i: (i, 0)),
            pl.BlockSpec((R, D), lambda i: (i, 0)),
        ],
        out_shape=[jax.ShapeDtypeStruct((N, D), jnp.float32)] * 2,
    )


def _tc_combine_transform():
    def body(p0_ref, p1_ref, c0_ref, c1_ref, r_ref, m_ref, wl_ref, wr_ref,
             b_ref, xp_ref, rout_ref):
        cnt = c0_ref[:, 0:1] + c1_ref[:, 0:1]
        mean = (p0_ref[...] + p1_ref[...]) / jnp.maximum(cnt, 1.0)
        h = jnp.maximum((mean + r_ref[...]) * m_ref[...], 0.0)
        dn = (((1,), (1,)), ((), ()))
        xp_ref[...] = lax.dot_general(h, wl_ref[...], dn,
                                      preferred_element_type=jnp.float32)
        rout_ref[...] = lax.dot_general(h, wr_ref[...], dn,
                                        preferred_element_type=jnp.float32) + b_ref[...]

    return pl.pallas_call(
        body,
        grid=(N // R,),
        in_specs=[
            pl.BlockSpec((R, D), lambda i: (i, 0)),
            pl.BlockSpec((R, D), lambda i: (i, 0)),
            pl.BlockSpec((R, LANES), lambda i: (i, 0)),
            pl.BlockSpec((R, LANES), lambda i: (i, 0)),
            pl.BlockSpec((R, D), lambda i: (i, 0)),
            pl.BlockSpec((R, D), lambda i: (i, 0)),
            pl.BlockSpec((D, D), lambda i: (0, 0)),
            pl.BlockSpec((D, D), lambda i: (0, 0)),
            pl.BlockSpec((1, D), lambda i: (0, 0)),
        ],
        out_specs=[
            pl.BlockSpec((R, D), lambda i: (i, 0)),
            pl.BlockSpec((R, D), lambda i: (i, 0)),
        ],
        out_shape=[jax.ShapeDtypeStruct((N, D), jnp.float32)] * 2,
    )


def _tc_final():
    def body(p0_ref, p1_ref, c0_ref, c1_ref, r_ref, o_ref):
        cnt = c0_ref[:, 0:1] + c1_ref[:, 0:1]
        mean = (p0_ref[...] + p1_ref[...]) / jnp.maximum(cnt, 1.0)
        o_ref[...] = mean + r_ref[...]

    return pl.pallas_call(
        body,
        grid=(N // R,),
        in_specs=[
            pl.BlockSpec((R, D), lambda i: (i, 0)),
            pl.BlockSpec((R, D), lambda i: (i, 0)),
            pl.BlockSpec((R, LANES), lambda i: (i, 0)),
            pl.BlockSpec((R, LANES), lambda i: (i, 0)),
            pl.BlockSpec((R, D), lambda i: (i, 0)),
        ],
        out_specs=pl.BlockSpec((R, D), lambda i: (i, 0)),
        out_shape=jax.ShapeDtypeStruct((N, D), jnp.float32),
    )


def _zero_vmem_2d(buf, rows, cols):
    def zrow(i, carry):
        for j in range(cols // LANES):
            buf[i, pl.ds(j * LANES, LANES)] = jnp.zeros((LANES,), jnp.float32)
        return carry
    lax.fori_loop(0, rows, zrow, 0)


def _make_sc_aggregate():
    mesh = plsc.VectorSubcoreMesh(core_axis_name="c", subcore_axis_name="s")

    def body(xp_hbm, sd_hbm, out_hbm, sd_v, sb0, sb1, db, g0, g1,
             acc, sem0, sem1):
        c = lax.axis_index("c")
        s = lax.axis_index("s")
        g = c * NT + s
        base = s * RPT

        def dec_src(j, buf):
            for l in range(K2 // LANES):
                w = sd_v[j, pl.ds(l * LANES, LANES)]
                buf[pl.ds(l * LANES, LANES)] = w >> 16

        def dec_dst(j, buf):
            for l in range(K2 // LANES):
                w = sd_v[j, pl.ds(l * LANES, LANES)]
                buf[pl.ds(l * LANES, LANES)] = w & 0xFFFF

        _zero_vmem_2d(g0, K2, D)
        for t in range(RPT // K2):
            pltpu.sync_copy(g0, acc.at[pl.ds(base + t * K2, K2)])
        pltpu.sync_copy(sd_hbm.at[g], sd_v)
        plsc.subcore_barrier()

        dec_src(0, sb0)
        pltpu.async_copy(xp_hbm.at[sb0], g0, sem0)

        def pair(t, carry):
            a = 2 * t
            b = a + 1
            dec_src(b, sb1)
            pltpu.async_copy(xp_hbm.at[sb1], g1, sem1)
            pltpu.make_async_copy(xp_hbm.at[sb0], g0, sem0).wait()
            dec_dst(a, db)
            pltpu.sync_copy(g0, acc.at[db], add=True)
            dec_src(b + 1, sb0)
            pltpu.async_copy(xp_hbm.at[sb0], g0, sem0)
            pltpu.make_async_copy(xp_hbm.at[sb1], g1, sem1).wait()
            dec_dst(b, db)
            pltpu.sync_copy(g1, acc.at[db], add=True)
            return carry
        lax.fori_loop(0, (CH2 - 1) // 2, pair, 0)

        last = CH2 - 1
        pltpu.make_async_copy(xp_hbm.at[sb0], g0, sem0).wait()
        dec_dst(last, db)
        pltpu.sync_copy(g0, acc.at[db], add=True)

        plsc.subcore_barrier()

        for t in range(RPT // ZR):
            rows = pl.ds(base + t * ZR, ZR)
            pltpu.sync_copy(acc.at[rows], out_hbm.at[c].at[rows])

    return pl.kernel(
        body,
        out_type=jax.ShapeDtypeStruct((NSC, N2, D), jnp.float32),
        mesh=mesh,
        compiler_params=pltpu.CompilerParams(use_tc_tiling_on_sc=False),
        scratch_types=[
            pltpu.VMEM((CH2, K2), jnp.int32),
            pltpu.VMEM((K2,), jnp.int32),
            pltpu.VMEM((K2,), jnp.int32),
            pltpu.VMEM((K2,), jnp.int32),
            pltpu.VMEM((K2, D), jnp.float32),
            pltpu.VMEM((K2, D), jnp.float32),
            pltpu.VMEM_SHARED((N2, D), jnp.float32),
            pltpu.SemaphoreType.DMA,
            pltpu.SemaphoreType.DMA,
        ],
    )


def _make_sc_counts():
    mesh = plsc.VectorSubcoreMesh(core_axis_name="c", subcore_axis_name="s")

    def body(dst_hbm, out_hbm, dst_v, obuf, zbuf, cacc, sem):
        c = lax.axis_index("c")
        s = lax.axis_index("s")
        g = c * NT + s
        base = s * RPT

        _zero_vmem_2d(zbuf, ZR, LANES)
        for t in range(RPT // ZR):
            pltpu.sync_copy(zbuf, cacc.at[pl.ds(base + t * ZR, ZR)])

        def orow(i, carry):
            obuf[i, :] = jnp.ones((LANES,), jnp.float32)
            return carry
        lax.fori_loop(0, K, orow, 0)
        pltpu.sync_copy(dst_hbm.at[g], dst_v)
        plsc.subcore_barrier()

        def fire(j, carry):
            pltpu.async_copy(obuf, cacc.at[dst_v.at[j]], sem, add=True)
            return carry
        lax.fori_loop(0, CH, fire, 0)

        def drain(j, carry):
            pltpu.make_async_copy(obuf, cacc.at[dst_v.at[0]], sem).wait()
            return carry
        lax.fori_loop(0, CH, drain, 0)

        plsc.subcore_barrier()

        for t in range(RPT // ZR):
            rows = pl.ds(base + t * ZR, ZR)
            pltpu.sync_copy(cacc.at[rows], out_hbm.at[c].at[rows])

    return pl.kernel(
        body,
        out_type=jax.ShapeDtypeStruct((NSC, N2, LANES), jnp.float32),
        mesh=mesh,
        compiler_params=pltpu.CompilerParams(use_tc_tiling_on_sc=False),
        scratch_types=[
            pltpu.VMEM((CH, K), jnp.int32),
            pltpu.VMEM((K, LANES), jnp.float32),
            pltpu.VMEM((ZR, LANES), jnp.float32),
            pltpu.VMEM_SHARED((N2, LANES), jnp.float32),
            pltpu.SemaphoreType.DMA,
        ],
    )


def _np_threefry2x32(k1, k2, x0, x1):
    rot0 = (13, 15, 26, 6)
    rot1 = (17, 29, 16, 24)
    ks = [np.uint32(k1), np.uint32(k2),
          np.uint32(k1) ^ np.uint32(k2) ^ np.uint32(0x1BD11BDA)]
    x = [x0.astype(np.uint32), x1.astype(np.uint32)]

    def rotl(v, d):
        return (v << np.uint32(d)) | (v >> np.uint32(32 - d))

    def rounds(x, rots):
        for r in rots:
            x[0] = x[0] + x[1]
            x[1] = x[0] ^ rotl(x[1], r)
        return x

    x[0] = x[0] + ks[0]
    x[1] = x[1] + ks[1]
    for i, (rots, ka, kb) in enumerate(
            ((rot0, 1, 2), (rot1, 2, 0), (rot0, 0, 1),
             (rot1, 1, 2), (rot0, 2, 0))):
        x = rounds(x, rots)
        x[0] = x[0] + ks[ka]
        x[1] = x[1] + ks[kb] + np.uint32(i + 1)
    return x


def _dropout_scale(seed):
    n = N * D
    with np.errstate(over="ignore"):
        b1, b2 = _np_threefry2x32(0, seed, np.zeros((n,), np.uint32),
                                  np.arange(n, dtype=np.uint32))
        bits = (b1 ^ b2).reshape(N, D)
        fb = (bits >> np.uint32(9)) | np.uint32(0x3F800000)
        u = fb.view(np.float32) - np.float32(1.0)
    return (u < np.float32(0.5)).astype(np.float32) * np.float32(2.0)


_M1 = _dropout_scale(101)
_M2 = _dropout_scale(102)


def kernel(x, edge_index, Wl1, Wr1, b1, Wl2, Wr2, b2, Wl3, Wr3, b3):
    dst3 = edge_index[1].reshape(NW, CH, K)
    sd = (edge_index[0] << 16) | edge_index[1]
    sdp = jnp.full((E2,), (N2 - 1), dtype=jnp.int32).at[:E].set(sd)
    sdp = sdp.reshape(NW, CH2, K2)
    b1r = b1.reshape(1, D)
    b2r = b2.reshape(1, D)
    b3r = b3.reshape(1, D)

    m1 = _M1
    m2 = _M2

    sc_counts = _make_sc_counts()
    sc_agg = _make_sc_aggregate()
    tc_transform = _tc_transform()
    tc_mid = _tc_combine_transform()
    tc_final = _tc_final()

    cnt = sc_counts(dst3)
    c0, c1 = cnt[0], cnt[1]
    xp1, r1 = tc_transform(x, Wl1, Wr1, b1r)
    p = sc_agg(xp1, sdp)
    xp2, r2 = tc_mid(p[0], p[1], c0, c1, r1, m1, Wl2, Wr2, b2r)
    p = sc_agg(xp2, sdp)
    xp3, r3 = tc_mid(p[0], p[1], c0, c1, r2, m2, Wl3, Wr3, b3r)
    p = sc_agg(xp3, sdp)
    return tc_final(p[0], p[1], c0, c1, r3)


# speedup vs baseline: 1.8845x; 1.8845x over previous
import jax
import jax.numpy as jnp
import numpy as np
from jax import lax
from jax.experimental import pallas as pl
from jax.experimental.pallas import tpu as pltpu
from jax.experimental.pallas import tpu_sc as plsc

N = 10000
D = 128
E = 320000

NSC = 2
NT = 16
NW = NSC * NT
EPT = E // NW
K = 80
CH = EPT // K
N2 = 10240
RPT = N2 // NT
ZR = 128
LANES = 16
R = 2000


def _tc_transform():
    def body(x_ref, wl_ref, wr_ref, b_ref, xp_ref, r_ref):
        xb = x_ref[...]
        dn = (((1,), (1,)), ((), ()))
        xp_ref[...] = lax.dot_general(xb, wl_ref[...], dn,
                                      preferred_element_type=jnp.float32)
        r_ref[...] = lax.dot_general(xb, wr_ref[...], dn,
                                     preferred_element_type=jnp.float32) + b_ref[...]

    return pl.pallas_call(
        body,
        grid=(N // R,),
        in_specs=[
            pl.BlockSpec((R, D), lambda i: (i, 0)),
            pl.BlockSpec((D, D), lambda i: (0, 0)),
            pl.BlockSpec((D, D), lambda i: (0, 0)),
            pl.BlockSpec((1, D), lambda i: (0, 0)),
        ],
        out_specs=[
            pl.BlockSpec((R, D), lambda i: (i, 0)),
            pl.BlockSpec((R, D), lambda i: (i, 0)),
        ],
        out_shape=[jax.ShapeDtypeStruct((N, D), jnp.float32)] * 2,
    )


def _tc_combine_transform():
    def body(p0_ref, p1_ref, c0_ref, c1_ref, r_ref, m_ref, wl_ref, wr_ref,
             b_ref, xp_ref, rout_ref):
        cnt = c0_ref[:, 0:1] + c1_ref[:, 0:1]
        mean = (p0_ref[...] + p1_ref[...]) / jnp.maximum(cnt, 1.0)
        h = jnp.maximum((mean + r_ref[...]) * m_ref[...], 0.0)
        dn = (((1,), (1,)), ((), ()))
        xp_ref[...] = lax.dot_general(h, wl_ref[...], dn,
                                      preferred_element_type=jnp.float32)
        rout_ref[...] = lax.dot_general(h, wr_ref[...], dn,
                                        preferred_element_type=jnp.float32) + b_ref[...]

    return pl.pallas_call(
        body,
        grid=(N // R,),
        in_specs=[
            pl.BlockSpec((R, D), lambda i: (i, 0)),
            pl.BlockSpec((R, D), lambda i: (i, 0)),
            pl.BlockSpec((R, LANES), lambda i: (i, 0)),
            pl.BlockSpec((R, LANES), lambda i: (i, 0)),
            pl.BlockSpec((R, D), lambda i: (i, 0)),
            pl.BlockSpec((R, D), lambda i: (i, 0)),
            pl.BlockSpec((D, D), lambda i: (0, 0)),
            pl.BlockSpec((D, D), lambda i: (0, 0)),
            pl.BlockSpec((1, D), lambda i: (0, 0)),
        ],
        out_specs=[
            pl.BlockSpec((R, D), lambda i: (i, 0)),
            pl.BlockSpec((R, D), lambda i: (i, 0)),
        ],
        out_shape=[jax.ShapeDtypeStruct((N, D), jnp.float32)] * 2,
    )


def _tc_final():
    def body(p0_ref, p1_ref, c0_ref, c1_ref, r_ref, o_ref):
        cnt = c0_ref[:, 0:1] + c1_ref[:, 0:1]
        mean = (p0_ref[...] + p1_ref[...]) / jnp.maximum(cnt, 1.0)
        o_ref[...] = mean + r_ref[...]

    return pl.pallas_call(
        body,
        grid=(N // R,),
        in_specs=[
            pl.BlockSpec((R, D), lambda i: (i, 0)),
            pl.BlockSpec((R, D), lambda i: (i, 0)),
            pl.BlockSpec((R, LANES), lambda i: (i, 0)),
            pl.BlockSpec((R, LANES), lambda i: (i, 0)),
            pl.BlockSpec((R, D), lambda i: (i, 0)),
        ],
        out_specs=pl.BlockSpec((R, D), lambda i: (i, 0)),
        out_shape=jax.ShapeDtypeStruct((N, D), jnp.float32),
    )


def _zero_vmem_2d(buf, rows, cols):
    def zrow(i, carry):
        for j in range(cols // LANES):
            buf[i, pl.ds(j * LANES, LANES)] = jnp.zeros((LANES,), jnp.float32)
        return carry
    lax.fori_loop(0, rows, zrow, 0)


def _make_sc_aggregate():
    mesh = plsc.VectorSubcoreMesh(core_axis_name="c", subcore_axis_name="s")

    def body(xp_hbm, src_hbm, dst_hbm, out_hbm, src_v, dst_v, g0, g1,
             acc, sem0, sem1):
        c = lax.axis_index("c")
        s = lax.axis_index("s")
        g = c * NT + s
        base = s * RPT

        _zero_vmem_2d(g0, K, D)
        for t in range(RPT // K):
            pltpu.sync_copy(g0, acc.at[pl.ds(base + t * K, K)])
        pltpu.sync_copy(src_hbm.at[g], src_v)
        pltpu.sync_copy(dst_hbm.at[g], dst_v)
        plsc.subcore_barrier()

        pltpu.async_copy(xp_hbm.at[src_v.at[0]], g0, sem0)

        def pair(t, carry):
            a = 2 * t
            b = a + 1
            pltpu.async_copy(xp_hbm.at[src_v.at[b]], g1, sem1)
            pltpu.make_async_copy(xp_hbm.at[src_v.at[a]], g0, sem0).wait()
            pltpu.sync_copy(g0, acc.at[dst_v.at[a]], add=True)
            pltpu.async_copy(xp_hbm.at[src_v.at[b + 1]], g0, sem0)
            pltpu.make_async_copy(xp_hbm.at[src_v.at[b]], g1, sem1).wait()
            pltpu.sync_copy(g1, acc.at[dst_v.at[b]], add=True)
            return carry
        lax.fori_loop(0, (CH - 1) // 2, pair, 0)

        last = CH - 1
        pltpu.make_async_copy(xp_hbm.at[src_v.at[last]], g0, sem0).wait()
        pltpu.sync_copy(g0, acc.at[dst_v.at[last]], add=True)

        plsc.subcore_barrier()

        for t in range(RPT // ZR):
            rows = pl.ds(base + t * ZR, ZR)
            pltpu.sync_copy(acc.at[rows], out_hbm.at[c].at[rows])

    return pl.kernel(
        body,
        out_type=jax.ShapeDtypeStruct((NSC, N2, D), jnp.float32),
        mesh=mesh,
        compiler_params=pltpu.CompilerParams(use_tc_tiling_on_sc=False),
        scratch_types=[
            pltpu.VMEM((CH, K), jnp.int32),
            pltpu.VMEM((CH, K), jnp.int32),
            pltpu.VMEM((K, D), jnp.float32),
            pltpu.VMEM((K, D), jnp.float32),
            pltpu.VMEM_SHARED((N2, D), jnp.float32),
            pltpu.SemaphoreType.DMA,
            pltpu.SemaphoreType.DMA,
        ],
    )


def _make_sc_counts():
    mesh = plsc.VectorSubcoreMesh(core_axis_name="c", subcore_axis_name="s")

    def body(dst_hbm, out_hbm, dst_v, obuf, zbuf, cacc, sem):
        c = lax.axis_index("c")
        s = lax.axis_index("s")
        g = c * NT + s
        base = s * RPT

        _zero_vmem_2d(zbuf, ZR, LANES)
        for t in range(RPT // ZR):
            pltpu.sync_copy(zbuf, cacc.at[pl.ds(base + t * ZR, ZR)])

        def orow(i, carry):
            obuf[i, :] = jnp.ones((LANES,), jnp.float32)
            return carry
        lax.fori_loop(0, K, orow, 0)
        pltpu.sync_copy(dst_hbm.at[g], dst_v)
        plsc.subcore_barrier()

        def fire(j, carry):
            pltpu.async_copy(obuf, cacc.at[dst_v.at[j]], sem, add=True)
            return carry
        lax.fori_loop(0, CH, fire, 0)

        def drain(j, carry):
            pltpu.make_async_copy(obuf, cacc.at[dst_v.at[0]], sem).wait()
            return carry
        lax.fori_loop(0, CH, drain, 0)

        plsc.subcore_barrier()

        for t in range(RPT // ZR):
            rows = pl.ds(base + t * ZR, ZR)
            pltpu.sync_copy(cacc.at[rows], out_hbm.at[c].at[rows])

    return pl.kernel(
        body,
        out_type=jax.ShapeDtypeStruct((NSC, N2, LANES), jnp.float32),
        mesh=mesh,
        compiler_params=pltpu.CompilerParams(use_tc_tiling_on_sc=False),
        scratch_types=[
            pltpu.VMEM((CH, K), jnp.int32),
            pltpu.VMEM((K, LANES), jnp.float32),
            pltpu.VMEM((ZR, LANES), jnp.float32),
            pltpu.VMEM_SHARED((N2, LANES), jnp.float32),
            pltpu.SemaphoreType.DMA,
        ],
    )


def _np_threefry2x32(k1, k2, x0, x1):
    rot0 = (13, 15, 26, 6)
    rot1 = (17, 29, 16, 24)
    ks = [np.uint32(k1), np.uint32(k2),
          np.uint32(k1) ^ np.uint32(k2) ^ np.uint32(0x1BD11BDA)]
    x = [x0.astype(np.uint32), x1.astype(np.uint32)]

    def rotl(v, d):
        return (v << np.uint32(d)) | (v >> np.uint32(32 - d))

    def rounds(x, rots):
        for r in rots:
            x[0] = x[0] + x[1]
            x[1] = x[0] ^ rotl(x[1], r)
        return x

    x[0] = x[0] + ks[0]
    x[1] = x[1] + ks[1]
    for i, (rots, ka, kb) in enumerate(
            ((rot0, 1, 2), (rot1, 2, 0), (rot0, 0, 1),
             (rot1, 1, 2), (rot0, 2, 0))):
        x = rounds(x, rots)
        x[0] = x[0] + ks[ka]
        x[1] = x[1] + ks[kb] + np.uint32(i + 1)
    return x


def _dropout_scale(seed):
    n = N * D
    with np.errstate(over="ignore"):
        b1, b2 = _np_threefry2x32(0, seed, np.zeros((n,), np.uint32),
                                  np.arange(n, dtype=np.uint32))
        bits = (b1 ^ b2).reshape(N, D)
        fb = (bits >> np.uint32(9)) | np.uint32(0x3F800000)
        u = fb.view(np.float32) - np.float32(1.0)
    return (u < np.float32(0.5)).astype(np.float32) * np.float32(2.0)


_M1 = _dropout_scale(101)
_M2 = _dropout_scale(102)


def kernel(x, edge_index, Wl1, Wr1, b1, Wl2, Wr2, b2, Wl3, Wr3, b3):
    src3 = edge_index[0].reshape(NW, CH, K)
    dst3 = edge_index[1].reshape(NW, CH, K)
    b1r = b1.reshape(1, D)
    b2r = b2.reshape(1, D)
    b3r = b3.reshape(1, D)

    m1 = _M1
    m2 = _M2

    sc_counts = _make_sc_counts()
    sc_agg = _make_sc_aggregate()
    tc_transform = _tc_transform()
    tc_mid = _tc_combine_transform()
    tc_final = _tc_final()

    cnt = sc_counts(dst3)
    c0, c1 = cnt[0], cnt[1]
    xp1, r1 = tc_transform(x, Wl1, Wr1, b1r)
    p = sc_agg(xp1, src3, dst3)
    xp2, r2 = tc_mid(p[0], p[1], c0, c1, r1, m1, Wl2, Wr2, b2r)
    p = sc_agg(xp2, src3, dst3)
    xp3, r3 = tc_mid(p[0], p[1], c0, c1, r2, m2, Wl3, Wr3, b3r)
    p = sc_agg(xp3, src3, dst3)
    return tc_final(p[0], p[1], c0, c1, r3)
